# baseline (speedup 1.0000x reference)
import functools

import jax
import jax.numpy as jnp
from jax import lax
from jax.experimental import pallas as pl
from jax.experimental.pallas import tpu as pltpu
from jax.experimental.pallas import tpu_sc as plsc

BM = 512
CR = 16
CT = 16
UNROLL = 8

def _sc_dims():
    try:
        info = plsc.get_sparse_core_info()
        return info.num_cores, info.num_subcores
    except Exception:
        return 2, 16

_NC, _NS = _sc_dims()
_NW = _NC * _NS


def _routing_metadata(top_k_index, E, n_blocks):
    T, K = top_k_index.shape
    flat_e = top_k_index.reshape(-1).astype(jnp.int32)
    onehot = (flat_e[:, None] == jnp.arange(E, dtype=jnp.int32)[None, :])
    onehot = onehot.astype(jnp.int32)
    csum = jnp.cumsum(onehot, axis=0)
    counts = csum[-1]
    rank = jnp.sum(onehot * csum, axis=1) - 1
    padded = ((counts + BM - 1) // BM) * BM
    p_off = jnp.concatenate(
        [jnp.zeros((1,), jnp.int32), jnp.cumsum(padded)[:-1].astype(jnp.int32)])
    dest = jnp.sum(onehot * p_off[None, :], axis=1).astype(jnp.int32) + rank
    blk_end = (jnp.cumsum(padded) // BM).astype(jnp.int32)
    blk_ids = jnp.arange(n_blocks, dtype=jnp.int32)
    block_expert = jnp.sum(
        (blk_end[None, :] <= blk_ids[:, None]).astype(jnp.int32), axis=1)
    block_expert = jnp.minimum(block_expert, E - 1)
    return dest, block_expert


def _make_dispatch(T, D, K, N_pad):
    mesh = plsc.VectorSubcoreMesh(core_axis_name="c", subcore_axis_name="s")
    tpw = T // _NW

    @functools.partial(
        pl.kernel, mesh=mesh,
        out_type=jax.ShapeDtypeStruct((N_pad, D), jnp.float32),
        scratch_types=[
            pltpu.VMEM((CR, D), jnp.float32),
            pltpu.VMEM((1, CR), jnp.int32),
            pltpu.SemaphoreType.DMA,
        ],
    )
    def dispatch(hs_hbm, dslot_hbm, xpad_hbm, xrows, dbuf, sem):
        wid = lax.axis_index("s") * _NC + lax.axis_index("c")
        for s in range(K):
            @pl.loop(0, tpw // CR)
            def _(j, s=s):
                base = wid * tpw + j * CR
                pltpu.sync_copy(hs_hbm.at[pl.ds(base, CR)], xrows)
                pltpu.sync_copy(dslot_hbm.at[s, pl.ds(base, CR)], dbuf.at[0])
                pltpu.async_copy(xrows, xpad_hbm.at[dbuf.at[0]], sem).wait()

    return dispatch


def _make_combine(T, D, K, N_pad):
    mesh = plsc.VectorSubcoreMesh(core_axis_name="c", subcore_axis_name="s")
    tpw = T // _NW

    scratch = [pltpu.VMEM((CT, D), jnp.float32) for _ in range(K)]
    scratch += [pltpu.VMEM((1, CT), jnp.int32) for _ in range(K)]
    scratch += [pltpu.VMEM((CT * K, 16), jnp.float32), pltpu.SemaphoreType.DMA]

    @functools.partial(
        pl.kernel, mesh=mesh,
        out_type=jax.ShapeDtypeStruct((T, D), jnp.float32),
        scratch_types=scratch,
    )
    def combine(y_hbm, dslot_hbm, w_hbm, out_hbm, *rest):
        rbufs = rest[:K]
        ibufs = rest[K:2 * K]
        wbuf = rest[2 * K]
        sem = rest[2 * K + 1]
        wid = lax.axis_index("s") * _NC + lax.axis_index("c")

        @pl.loop(0, tpw // CT)
        def _(j):
            base = wid * tpw + j * CT
            for s in range(K):
                pltpu.sync_copy(dslot_hbm.at[s, pl.ds(base, CT)], ibufs[s].at[0])
            pltpu.sync_copy(w_hbm.at[pl.ds(base * K, CT * K)], wbuf)
            copies = [pltpu.async_copy(y_hbm.at[ibufs[s].at[0]], rbufs[s], sem)
                      for s in range(K)]
            for c in copies:
                c.wait()

            @pl.loop(0, CT)
            def _(t):
                ws = [wbuf[t * K + s] for s in range(K)]

                @pl.loop(0, D, step=16 * UNROLL)
                def _(v):
                    for u in range(UNROLL):
                        sl = pl.ds(v + u * 16, 16)
                        acc = ws[0] * rbufs[0][t, sl]
                        for s in range(1, K):
                            acc = acc + ws[s] * rbufs[s][t, sl]
                        rbufs[0][t, sl] = acc

            pltpu.sync_copy(rbufs[0], out_hbm.at[pl.ds(base, CT)])

    return combine


_DIMS_NT = (((1,), (1,)), ((), ()))


def _run_metadata(block_expert, n_blocks):
    be = block_expert
    chg = jnp.concatenate(
        [jnp.ones((1,), jnp.int32), (be[1:] != be[:-1]).astype(jnp.int32)])
    rid = jnp.cumsum(chg) - 1
    run_expert = jnp.zeros((n_blocks + 1,), jnp.int32).at[rid].set(be)
    n_runs = rid[-1] + 1
    nxe = jnp.where(rid + 1 < n_runs, run_expert[rid + 1], -1)
    return rid.astype(jnp.int32), nxe.astype(jnp.int32)


def _slab(w_any, e, half, span):
    if half is None:
        return w_any.at[e]
    return w_any.at[e, pl.ds(half * span, span)]


def _stage_weights(refs, w_any, half, span):
    be_ref, rid_ref, nxe_ref, stg_ref, wbf_ref, sems = refs
    i = pl.program_id(0)
    rid = rid_ref[i]
    par = lax.rem(rid, 2)
    prev_rid = rid_ref[jnp.maximum(i - 1, 0)]
    is_start = (i == 0) | (rid != prev_rid)

    @pl.when(i == 0)
    def _():
        pltpu.make_async_copy(_slab(w_any, be_ref[0], half, span),
                              stg_ref.at[0], sems.at[0]).start()

    @pl.when(is_start)
    def _():
        pltpu.make_async_copy(_slab(w_any, be_ref[i], half, span),
                              stg_ref.at[par], sems.at[par]).wait()
        wbf_ref[...] = stg_ref[par].astype(jnp.bfloat16)

        @pl.when(nxe_ref[i] >= 0)
        def _():
            pltpu.make_async_copy(_slab(w_any, nxe_ref[i], half, span),
                                  stg_ref.at[1 - par], sems.at[1 - par]).start()


def _gate_body(be_ref, rid_ref, nxe_ref, x_ref, w_any, sg_ref, xbf_ref,
               stg_ref, wbf_ref, sems):
    _stage_weights((be_ref, rid_ref, nxe_ref, stg_ref, wbf_ref, sems),
                   w_any, 0, wbf_ref.shape[0])
    x = x_ref[...].astype(jnp.bfloat16)
    xbf_ref[...] = x
    g = lax.dot_general(x, wbf_ref[...], _DIMS_NT,
                        preferred_element_type=jnp.float32)
    sg_ref[...] = (g * jax.nn.sigmoid(g)).astype(jnp.bfloat16)


def _up_body(be_ref, rid_ref, nxe_ref, xbf_ref, w_any, sg_ref, h_ref,
             stg_ref, wbf_ref, sems):
    _stage_weights((be_ref, rid_ref, nxe_ref, stg_ref, wbf_ref, sems),
                   w_any, 1, wbf_ref.shape[0])
    up = lax.dot_general(xbf_ref[...], wbf_ref[...], _DIMS_NT,
                         preferred_element_type=jnp.float32)
    h_ref[...] = (up * sg_ref[...].astype(jnp.float32)).astype(jnp.bfloat16)


def _down_body(be_ref, rid_ref, nxe_ref, h_ref, w_any, y_ref,
               stg_ref, wbf_ref, sems):
    _stage_weights((be_ref, rid_ref, nxe_ref, stg_ref, wbf_ref, sems),
                   w_any, None, None)
    y_ref[...] = lax.dot_general(h_ref[...], wbf_ref[...], _DIMS_NT,
                                 preferred_element_type=jnp.float32)


def kernel(hidden_states, top_k_index, top_k_weights, gate_up_proj, down_proj):
    T, D = hidden_states.shape
    K = top_k_index.shape[1]
    E, I2, _ = gate_up_proj.shape
    I = I2 // 2
    N_pad = T * K + E * BM
    n_blocks = N_pad // BM

    dest, block_expert = _routing_metadata(top_k_index, E, n_blocks)
    dest_slots = dest.reshape(T, K).T

    x_pad = _make_dispatch(T, D, K, N_pad)(hidden_states, dest_slots)

    rid, nxe = _run_metadata(block_expert, n_blocks)

    grid_g = pltpu.PrefetchScalarGridSpec(
        num_scalar_prefetch=3,
        grid=(n_blocks,),
        in_specs=[
            pl.BlockSpec((BM, D), lambda i, be, rid, nxe: (i, 0)),
            pl.BlockSpec(memory_space=pl.ANY),
        ],
        out_specs=[
            pl.BlockSpec((BM, I), lambda i, be, rid, nxe: (i, 0)),
            pl.BlockSpec((BM, D), lambda i, be, rid, nxe: (i, 0)),
        ],
        scratch_shapes=[pltpu.VMEM((2, I, D), jnp.float32),
                        pltpu.VMEM((I, D), jnp.bfloat16),
                        pltpu.SemaphoreType.DMA((2,))],
    )
    sg_pad, xbf_pad = pl.pallas_call(
        _gate_body,
        grid_spec=grid_g,
        compiler_params=pltpu.CompilerParams(
            dimension_semantics=("parallel",)),
        out_shape=[jax.ShapeDtypeStruct((N_pad, I), jnp.bfloat16),
                   jax.ShapeDtypeStruct((N_pad, D), jnp.bfloat16)],
    )(block_expert, rid, nxe, x_pad, gate_up_proj)

    grid_u = pltpu.PrefetchScalarGridSpec(
        num_scalar_prefetch=3,
        grid=(n_blocks,),
        in_specs=[
            pl.BlockSpec((BM, D), lambda i, be, rid, nxe: (i, 0)),
            pl.BlockSpec(memory_space=pl.ANY),
            pl.BlockSpec((BM, I), lambda i, be, rid, nxe: (i, 0)),
        ],
        out_specs=pl.BlockSpec((BM, I), lambda i, be, rid, nxe: (i, 0)),
        scratch_shapes=[pltpu.VMEM((2, I, D), jnp.float32),
                        pltpu.VMEM((I, D), jnp.bfloat16),
                        pltpu.SemaphoreType.DMA((2,))],
    )
    h_pad = pl.pallas_call(
        _up_body,
        grid_spec=grid_u,
        compiler_params=pltpu.CompilerParams(
            dimension_semantics=("parallel",)),
        out_shape=jax.ShapeDtypeStruct((N_pad, I), jnp.bfloat16),
    )(block_expert, rid, nxe, xbf_pad, gate_up_proj, sg_pad)

    grid_d = pltpu.PrefetchScalarGridSpec(
        num_scalar_prefetch=3,
        grid=(n_blocks,),
        in_specs=[
            pl.BlockSpec((BM, I), lambda i, be, rid, nxe: (i, 0)),
            pl.BlockSpec(memory_space=pl.ANY),
        ],
        out_specs=pl.BlockSpec((BM, D), lambda i, be, rid, nxe: (i, 0)),
        scratch_shapes=[pltpu.VMEM((2, D, I), jnp.float32),
                        pltpu.VMEM((D, I), jnp.bfloat16),
                        pltpu.SemaphoreType.DMA((2,))],
    )
    y_pad = pl.pallas_call(
        _down_body,
        grid_spec=grid_d,
        compiler_params=pltpu.CompilerParams(
            dimension_semantics=("parallel",)),
        out_shape=jax.ShapeDtypeStruct((N_pad, D), jnp.float32),
    )(block_expert, rid, nxe, h_pad, down_proj)

    w_bc = jnp.broadcast_to(top_k_weights.reshape(-1)[:, None], (T * K, 16))
    return _make_combine(T, D, K, N_pad)(y_pad, dest_slots, w_bc)


# speedup vs baseline: 1.0509x; 1.0509x over previous
import functools

import jax
import jax.numpy as jnp
from jax import lax
from jax.experimental import pallas as pl
from jax.experimental.pallas import tpu as pltpu
from jax.experimental.pallas import tpu_sc as plsc

BM = 256
CR = 16
CT = 16
UNROLL = 8

def _sc_dims():
    try:
        info = plsc.get_sparse_core_info()
        return info.num_cores, info.num_subcores
    except Exception:
        return 2, 16

_NC, _NS = _sc_dims()
_NW = _NC * _NS


def _routing_metadata(top_k_index, E, n_blocks):
    T, K = top_k_index.shape
    flat_e = top_k_index.reshape(-1).astype(jnp.int32)
    onehot = (flat_e[:, None] == jnp.arange(E, dtype=jnp.int32)[None, :])
    onehot = onehot.astype(jnp.int32)
    csum = jnp.cumsum(onehot, axis=0)
    counts = csum[-1]
    rank = jnp.sum(onehot * csum, axis=1) - 1
    padded = ((counts + BM - 1) // BM) * BM
    p_off = jnp.concatenate(
        [jnp.zeros((1,), jnp.int32), jnp.cumsum(padded)[:-1].astype(jnp.int32)])
    dest = jnp.sum(onehot * p_off[None, :], axis=1).astype(jnp.int32) + rank
    blk_end = (jnp.cumsum(padded) // BM).astype(jnp.int32)
    blk_ids = jnp.arange(n_blocks, dtype=jnp.int32)
    block_expert = jnp.sum(
        (blk_end[None, :] <= blk_ids[:, None]).astype(jnp.int32), axis=1)
    block_expert = jnp.minimum(block_expert, E - 1)
    return dest, block_expert


def _make_dispatch(T, D, K, N_pad):
    mesh = plsc.VectorSubcoreMesh(core_axis_name="c", subcore_axis_name="s")
    tpw = T // _NW

    @functools.partial(
        pl.kernel, mesh=mesh,
        out_type=jax.ShapeDtypeStruct((N_pad, D), jnp.float32),
        scratch_types=[
            pltpu.VMEM((CR, D), jnp.float32),
            pltpu.VMEM((1, CR), jnp.int32),
            pltpu.SemaphoreType.DMA,
        ],
    )
    def dispatch(hs_hbm, dslot_hbm, xpad_hbm, xrows, dbuf, sem):
        wid = lax.axis_index("s") * _NC + lax.axis_index("c")
        for s in range(K):
            @pl.loop(0, tpw // CR)
            def _(j, s=s):
                base = wid * tpw + j * CR
                pltpu.sync_copy(hs_hbm.at[pl.ds(base, CR)], xrows)
                pltpu.sync_copy(dslot_hbm.at[s, pl.ds(base, CR)], dbuf.at[0])
                pltpu.async_copy(xrows, xpad_hbm.at[dbuf.at[0]], sem).wait()

    return dispatch


def _make_combine(T, D, K, N_pad):
    mesh = plsc.VectorSubcoreMesh(core_axis_name="c", subcore_axis_name="s")
    tpw = T // _NW

    scratch = [pltpu.VMEM((CT, D), jnp.float32) for _ in range(K)]
    scratch += [pltpu.VMEM((1, CT), jnp.int32) for _ in range(K)]
    scratch += [pltpu.VMEM((CT * K, 16), jnp.float32), pltpu.SemaphoreType.DMA]

    @functools.partial(
        pl.kernel, mesh=mesh,
        out_type=jax.ShapeDtypeStruct((T, D), jnp.float32),
        scratch_types=scratch,
    )
    def combine(y_hbm, dslot_hbm, w_hbm, out_hbm, *rest):
        rbufs = rest[:K]
        ibufs = rest[K:2 * K]
        wbuf = rest[2 * K]
        sem = rest[2 * K + 1]
        wid = lax.axis_index("s") * _NC + lax.axis_index("c")

        @pl.loop(0, tpw // CT)
        def _(j):
            base = wid * tpw + j * CT
            for s in range(K):
                pltpu.sync_copy(dslot_hbm.at[s, pl.ds(base, CT)], ibufs[s].at[0])
            pltpu.sync_copy(w_hbm.at[pl.ds(base * K, CT * K)], wbuf)
            copies = [pltpu.async_copy(y_hbm.at[ibufs[s].at[0]], rbufs[s], sem)
                      for s in range(K)]
            for c in copies:
                c.wait()

            @pl.loop(0, CT)
            def _(t):
                ws = [wbuf[t * K + s] for s in range(K)]

                @pl.loop(0, D, step=16 * UNROLL)
                def _(v):
                    for u in range(UNROLL):
                        sl = pl.ds(v + u * 16, 16)
                        acc = ws[0] * rbufs[0][t, sl]
                        for s in range(1, K):
                            acc = acc + ws[s] * rbufs[s][t, sl]
                        rbufs[0][t, sl] = acc

            pltpu.sync_copy(rbufs[0], out_hbm.at[pl.ds(base, CT)])

    return combine


_DIMS_NT = (((1,), (1,)), ((), ()))


def _run_metadata(block_expert, n_blocks):
    be = block_expert
    chg = jnp.concatenate(
        [jnp.ones((1,), jnp.int32), (be[1:] != be[:-1]).astype(jnp.int32)])
    rid = jnp.cumsum(chg) - 1
    run_expert = jnp.zeros((n_blocks + 1,), jnp.int32).at[rid].set(be)
    n_runs = rid[-1] + 1
    nxe = jnp.where(rid + 1 < n_runs, run_expert[rid + 1], -1)
    return rid.astype(jnp.int32), nxe.astype(jnp.int32)


def _slab(w_any, e, half, span):
    if half is None:
        return w_any.at[e]
    return w_any.at[e, pl.ds(half * span, span)]


def _stage_weights(refs, w_any, half, span):
    be_ref, rid_ref, nxe_ref, stg_ref, wbf_ref, sems = refs
    i = pl.program_id(0)
    rid = rid_ref[i]
    par = lax.rem(rid, 2)
    prev_rid = rid_ref[jnp.maximum(i - 1, 0)]
    is_start = (i == 0) | (rid != prev_rid)

    @pl.when(i == 0)
    def _():
        pltpu.make_async_copy(_slab(w_any, be_ref[0], half, span),
                              stg_ref.at[0], sems.at[0]).start()

    @pl.when(is_start)
    def _():
        pltpu.make_async_copy(_slab(w_any, be_ref[i], half, span),
                              stg_ref.at[par], sems.at[par]).wait()
        wbf_ref[...] = stg_ref[par].astype(jnp.bfloat16)

        @pl.when(nxe_ref[i] >= 0)
        def _():
            pltpu.make_async_copy(_slab(w_any, nxe_ref[i], half, span),
                                  stg_ref.at[1 - par], sems.at[1 - par]).start()


def _gate_body(be_ref, rid_ref, nxe_ref, x_ref, w_any, sg_ref, xbf_ref,
               stg_ref, wbf_ref, sems):
    _stage_weights((be_ref, rid_ref, nxe_ref, stg_ref, wbf_ref, sems),
                   w_any, 0, wbf_ref.shape[0])
    x = x_ref[...].astype(jnp.bfloat16)
    xbf_ref[...] = x
    g = lax.dot_general(x, wbf_ref[...], _DIMS_NT,
                        preferred_element_type=jnp.float32)
    sg_ref[...] = (g * jax.nn.sigmoid(g)).astype(jnp.bfloat16)


def _up_body(be_ref, rid_ref, nxe_ref, xbf_ref, w_any, sg_ref, h_ref,
             stg_ref, wbf_ref, sems):
    _stage_weights((be_ref, rid_ref, nxe_ref, stg_ref, wbf_ref, sems),
                   w_any, 1, wbf_ref.shape[0])
    up = lax.dot_general(xbf_ref[...], wbf_ref[...], _DIMS_NT,
                         preferred_element_type=jnp.float32)
    h_ref[...] = (up * sg_ref[...].astype(jnp.float32)).astype(jnp.bfloat16)


def _down_body(be_ref, rid_ref, nxe_ref, h_ref, w_any, y_ref,
               stg_ref, wbf_ref, sems):
    _stage_weights((be_ref, rid_ref, nxe_ref, stg_ref, wbf_ref, sems),
                   w_any, None, None)
    y_ref[...] = lax.dot_general(h_ref[...], wbf_ref[...], _DIMS_NT,
                                 preferred_element_type=jnp.float32)


def kernel(hidden_states, top_k_index, top_k_weights, gate_up_proj, down_proj):
    T, D = hidden_states.shape
    K = top_k_index.shape[1]
    E, I2, _ = gate_up_proj.shape
    I = I2 // 2
    N_pad = T * K + E * BM
    n_blocks = N_pad // BM

    dest, block_expert = _routing_metadata(top_k_index, E, n_blocks)
    dest_slots = dest.reshape(T, K).T

    x_pad = _make_dispatch(T, D, K, N_pad)(hidden_states, dest_slots)

    rid, nxe = _run_metadata(block_expert, n_blocks)

    grid_g = pltpu.PrefetchScalarGridSpec(
        num_scalar_prefetch=3,
        grid=(n_blocks,),
        in_specs=[
            pl.BlockSpec((BM, D), lambda i, be, rid, nxe: (i, 0)),
            pl.BlockSpec(memory_space=pl.ANY),
        ],
        out_specs=[
            pl.BlockSpec((BM, I), lambda i, be, rid, nxe: (i, 0)),
            pl.BlockSpec((BM, D), lambda i, be, rid, nxe: (i, 0)),
        ],
        scratch_shapes=[pltpu.VMEM((2, I, D), jnp.float32),
                        pltpu.VMEM((I, D), jnp.bfloat16),
                        pltpu.SemaphoreType.DMA((2,))],
    )
    sg_pad, xbf_pad = pl.pallas_call(
        _gate_body,
        grid_spec=grid_g,
        compiler_params=pltpu.CompilerParams(
            dimension_semantics=("parallel",)),
        out_shape=[jax.ShapeDtypeStruct((N_pad, I), jnp.bfloat16),
                   jax.ShapeDtypeStruct((N_pad, D), jnp.bfloat16)],
    )(block_expert, rid, nxe, x_pad, gate_up_proj)

    grid_u = pltpu.PrefetchScalarGridSpec(
        num_scalar_prefetch=3,
        grid=(n_blocks,),
        in_specs=[
            pl.BlockSpec((BM, D), lambda i, be, rid, nxe: (i, 0)),
            pl.BlockSpec(memory_space=pl.ANY),
            pl.BlockSpec((BM, I), lambda i, be, rid, nxe: (i, 0)),
        ],
        out_specs=pl.BlockSpec((BM, I), lambda i, be, rid, nxe: (i, 0)),
        scratch_shapes=[pltpu.VMEM((2, I, D), jnp.float32),
                        pltpu.VMEM((I, D), jnp.bfloat16),
                        pltpu.SemaphoreType.DMA((2,))],
    )
    h_pad = pl.pallas_call(
        _up_body,
        grid_spec=grid_u,
        compiler_params=pltpu.CompilerParams(
            dimension_semantics=("parallel",)),
        out_shape=jax.ShapeDtypeStruct((N_pad, I), jnp.bfloat16),
    )(block_expert, rid, nxe, xbf_pad, gate_up_proj, sg_pad)

    grid_d = pltpu.PrefetchScalarGridSpec(
        num_scalar_prefetch=3,
        grid=(n_blocks,),
        in_specs=[
            pl.BlockSpec((BM, I), lambda i, be, rid, nxe: (i, 0)),
            pl.BlockSpec(memory_space=pl.ANY),
        ],
        out_specs=pl.BlockSpec((BM, D), lambda i, be, rid, nxe: (i, 0)),
        scratch_shapes=[pltpu.VMEM((2, D, I), jnp.float32),
                        pltpu.VMEM((D, I), jnp.bfloat16),
                        pltpu.SemaphoreType.DMA((2,))],
    )
    y_pad = pl.pallas_call(
        _down_body,
        grid_spec=grid_d,
        compiler_params=pltpu.CompilerParams(
            dimension_semantics=("parallel",)),
        out_shape=jax.ShapeDtypeStruct((N_pad, D), jnp.float32),
    )(block_expert, rid, nxe, h_pad, down_proj)

    w_bc = jnp.broadcast_to(top_k_weights.reshape(-1)[:, None], (T * K, 16))
    return _make_combine(T, D, K, N_pad)(y_pad, dest_slots, w_bc)


# speedup vs baseline: 1.0723x; 1.0204x over previous
import functools

import jax
import jax.numpy as jnp
from jax import lax
from jax.experimental import pallas as pl
from jax.experimental.pallas import tpu as pltpu
from jax.experimental.pallas import tpu_sc as plsc

BM = 256
CR = 16
CT = 8
UNROLL = 8

def _sc_dims():
    try:
        info = plsc.get_sparse_core_info()
        return info.num_cores, info.num_subcores
    except Exception:
        return 2, 16

_NC, _NS = _sc_dims()
_NW = _NC * _NS


def _routing_metadata(top_k_index, E, n_blocks):
    T, K = top_k_index.shape
    flat_e = top_k_index.reshape(-1).astype(jnp.int32)
    onehot = (flat_e[:, None] == jnp.arange(E, dtype=jnp.int32)[None, :])
    onehot = onehot.astype(jnp.int32)
    csum = jnp.cumsum(onehot, axis=0)
    counts = csum[-1]
    rank = jnp.sum(onehot * csum, axis=1) - 1
    padded = ((counts + BM - 1) // BM) * BM
    p_off = jnp.concatenate(
        [jnp.zeros((1,), jnp.int32), jnp.cumsum(padded)[:-1].astype(jnp.int32)])
    dest = jnp.sum(onehot * p_off[None, :], axis=1).astype(jnp.int32) + rank
    blk_end = (jnp.cumsum(padded) // BM).astype(jnp.int32)
    blk_ids = jnp.arange(n_blocks, dtype=jnp.int32)
    block_expert = jnp.sum(
        (blk_end[None, :] <= blk_ids[:, None]).astype(jnp.int32), axis=1)
    block_expert = jnp.minimum(block_expert, E - 1)
    return dest, block_expert


def _make_dispatch(T, D, K, N_pad):
    mesh = plsc.VectorSubcoreMesh(core_axis_name="c", subcore_axis_name="s")
    tpw = T // _NW

    @functools.partial(
        pl.kernel, mesh=mesh,
        out_type=jax.ShapeDtypeStruct((N_pad, D), jnp.float32),
        scratch_types=[
            pltpu.VMEM((CR, D), jnp.float32),
            pltpu.VMEM((1, CR), jnp.int32),
            pltpu.SemaphoreType.DMA,
        ],
    )
    def dispatch(hs_hbm, dslot_hbm, xpad_hbm, xrows, dbuf, sem):
        wid = lax.axis_index("s") * _NC + lax.axis_index("c")
        for s in range(K):
            @pl.loop(0, tpw // CR)
            def _(j, s=s):
                base = wid * tpw + j * CR
                pltpu.sync_copy(hs_hbm.at[pl.ds(base, CR)], xrows)
                pltpu.sync_copy(dslot_hbm.at[s, pl.ds(base, CR)], dbuf.at[0])
                pltpu.async_copy(xrows, xpad_hbm.at[dbuf.at[0]], sem).wait()

    return dispatch


def _make_combine(T, D, K, N_pad):
    mesh = plsc.VectorSubcoreMesh(core_axis_name="c", subcore_axis_name="s")
    tpw = T // _NW
    nch = tpw // CT
    assert nch % 2 == 0

    scratch = [pltpu.VMEM((CT, D), jnp.float32) for _ in range(2 * K)]
    scratch += [pltpu.VMEM((tpw,), jnp.int32) for _ in range(K)]
    scratch += [pltpu.VMEM((tpw * K, 16), jnp.float32)]
    scratch += [pltpu.SemaphoreType.DMA, pltpu.SemaphoreType.DMA]

    @functools.partial(
        pl.kernel, mesh=mesh,
        out_type=jax.ShapeDtypeStruct((T, D), jnp.float32),
        scratch_types=scratch,
    )
    def combine(y_hbm, dslot_hbm, w_hbm, out_hbm, *rest):
        rbufs = [rest[:K], rest[K:2 * K]]
        ibufs = rest[2 * K:3 * K]
        wbuf = rest[3 * K]
        sems = [rest[3 * K + 1], rest[3 * K + 2]]
        wid = lax.axis_index("s") * _NC + lax.axis_index("c")
        base0 = wid * tpw

        for s in range(K):
            pltpu.sync_copy(dslot_hbm.at[s, pl.ds(base0, tpw)], ibufs[s])
        pltpu.sync_copy(w_hbm.at[pl.ds(base0 * K, tpw * K)], wbuf)

        def issue(j, par):
            for s in range(K):
                pltpu.async_copy(y_hbm.at[ibufs[s].at[pl.ds(j * CT, CT)]],
                                 rbufs[par][s], sems[par])

        def wait(j, par):
            for s in range(K):
                pltpu.make_async_copy(
                    y_hbm.at[ibufs[s].at[pl.ds(j * CT, CT)]],
                    rbufs[par][s], sems[par]).wait()

        def compute_store(j, par):
            bufs = rbufs[par]

            @pl.loop(0, CT)
            def _(t):
                woff = (j * CT + t) * K

                @pl.loop(0, D, step=16 * UNROLL)
                def _(v):
                    for u in range(UNROLL):
                        sl = pl.ds(v + u * 16, 16)
                        acc = wbuf[woff] * bufs[0][t, sl]
                        for s in range(1, K):
                            acc = acc + wbuf[woff + s] * bufs[s][t, sl]
                        bufs[0][t, sl] = acc

            pltpu.sync_copy(bufs[0], out_hbm.at[pl.ds(base0 + j * CT, CT)])

        issue(0, 0)

        @pl.loop(0, nch, step=2)
        def _(j):
            issue(j + 1, 1)
            wait(j, 0)
            compute_store(j, 0)

            @pl.when(j + 2 < nch)
            def _():
                issue(j + 2, 0)

            wait(j + 1, 1)
            compute_store(j + 1, 1)

    return combine


_DIMS_NT = (((1,), (1,)), ((), ()))


def _run_metadata(block_expert, n_blocks):
    be = block_expert
    chg = jnp.concatenate(
        [jnp.ones((1,), jnp.int32), (be[1:] != be[:-1]).astype(jnp.int32)])
    rid = jnp.cumsum(chg) - 1
    run_expert = jnp.zeros((n_blocks + 1,), jnp.int32).at[rid].set(be)
    n_runs = rid[-1] + 1
    nxe = jnp.where(rid + 1 < n_runs, run_expert[rid + 1], -1)
    return rid.astype(jnp.int32), nxe.astype(jnp.int32)


def _slab(w_any, e, half, span):
    if half is None:
        return w_any.at[e]
    return w_any.at[e, pl.ds(half * span, span)]


def _stage_weights(refs, w_any, half, span):
    be_ref, rid_ref, nxe_ref, stg_ref, wbf_ref, sems = refs
    i = pl.program_id(0)
    rid = rid_ref[i]
    par = lax.rem(rid, 2)
    prev_rid = rid_ref[jnp.maximum(i - 1, 0)]
    is_start = (i == 0) | (rid != prev_rid)

    @pl.when(i == 0)
    def _():
        pltpu.make_async_copy(_slab(w_any, be_ref[0], half, span),
                              stg_ref.at[0], sems.at[0]).start()

    @pl.when(is_start)
    def _():
        pltpu.make_async_copy(_slab(w_any, be_ref[i], half, span),
                              stg_ref.at[par], sems.at[par]).wait()
        wbf_ref[...] = stg_ref[par].astype(jnp.bfloat16)

        @pl.when(nxe_ref[i] >= 0)
        def _():
            pltpu.make_async_copy(_slab(w_any, nxe_ref[i], half, span),
                                  stg_ref.at[1 - par], sems.at[1 - par]).start()


def _gate_body(be_ref, rid_ref, nxe_ref, x_ref, w_any, sg_ref, xbf_ref,
               stg_ref, wbf_ref, sems):
    _stage_weights((be_ref, rid_ref, nxe_ref, stg_ref, wbf_ref, sems),
                   w_any, 0, wbf_ref.shape[0])
    x = x_ref[...].astype(jnp.bfloat16)
    xbf_ref[...] = x
    g = lax.dot_general(x, wbf_ref[...], _DIMS_NT,
                        preferred_element_type=jnp.float32)
    sg_ref[...] = (g * jax.nn.sigmoid(g)).astype(jnp.bfloat16)


def _up_body(be_ref, rid_ref, nxe_ref, xbf_ref, w_any, sg_ref, h_ref,
             stg_ref, wbf_ref, sems):
    _stage_weights((be_ref, rid_ref, nxe_ref, stg_ref, wbf_ref, sems),
                   w_any, 1, wbf_ref.shape[0])
    up = lax.dot_general(xbf_ref[...], wbf_ref[...], _DIMS_NT,
                         preferred_element_type=jnp.float32)
    h_ref[...] = (up * sg_ref[...].astype(jnp.float32)).astype(jnp.bfloat16)


def _down_body(be_ref, rid_ref, nxe_ref, h_ref, w_any, y_ref,
               stg_ref, wbf_ref, sems):
    _stage_weights((be_ref, rid_ref, nxe_ref, stg_ref, wbf_ref, sems),
                   w_any, None, None)
    y_ref[...] = lax.dot_general(h_ref[...], wbf_ref[...], _DIMS_NT,
                                 preferred_element_type=jnp.float32)


def kernel(hidden_states, top_k_index, top_k_weights, gate_up_proj, down_proj):
    T, D = hidden_states.shape
    K = top_k_index.shape[1]
    E, I2, _ = gate_up_proj.shape
    I = I2 // 2
    N_pad = T * K + E * BM
    n_blocks = N_pad // BM

    dest, block_expert = _routing_metadata(top_k_index, E, n_blocks)
    dest_slots = dest.reshape(T, K).T

    x_pad = _make_dispatch(T, D, K, N_pad)(hidden_states, dest_slots)

    rid, nxe = _run_metadata(block_expert, n_blocks)

    grid_g = pltpu.PrefetchScalarGridSpec(
        num_scalar_prefetch=3,
        grid=(n_blocks,),
        in_specs=[
            pl.BlockSpec((BM, D), lambda i, be, rid, nxe: (i, 0)),
            pl.BlockSpec(memory_space=pl.ANY),
        ],
        out_specs=[
            pl.BlockSpec((BM, I), lambda i, be, rid, nxe: (i, 0)),
            pl.BlockSpec((BM, D), lambda i, be, rid, nxe: (i, 0)),
        ],
        scratch_shapes=[pltpu.VMEM((2, I, D), jnp.float32),
                        pltpu.VMEM((I, D), jnp.bfloat16),
                        pltpu.SemaphoreType.DMA((2,))],
    )
    sg_pad, xbf_pad = pl.pallas_call(
        _gate_body,
        grid_spec=grid_g,
        compiler_params=pltpu.CompilerParams(
            dimension_semantics=("parallel",)),
        out_shape=[jax.ShapeDtypeStruct((N_pad, I), jnp.bfloat16),
                   jax.ShapeDtypeStruct((N_pad, D), jnp.bfloat16)],
    )(block_expert, rid, nxe, x_pad, gate_up_proj)

    grid_u = pltpu.PrefetchScalarGridSpec(
        num_scalar_prefetch=3,
        grid=(n_blocks,),
        in_specs=[
            pl.BlockSpec((BM, D), lambda i, be, rid, nxe: (i, 0)),
            pl.BlockSpec(memory_space=pl.ANY),
            pl.BlockSpec((BM, I), lambda i, be, rid, nxe: (i, 0)),
        ],
        out_specs=pl.BlockSpec((BM, I), lambda i, be, rid, nxe: (i, 0)),
        scratch_shapes=[pltpu.VMEM((2, I, D), jnp.float32),
                        pltpu.VMEM((I, D), jnp.bfloat16),
                        pltpu.SemaphoreType.DMA((2,))],
    )
    h_pad = pl.pallas_call(
        _up_body,
        grid_spec=grid_u,
        compiler_params=pltpu.CompilerParams(
            dimension_semantics=("parallel",)),
        out_shape=jax.ShapeDtypeStruct((N_pad, I), jnp.bfloat16),
    )(block_expert, rid, nxe, xbf_pad, gate_up_proj, sg_pad)

    grid_d = pltpu.PrefetchScalarGridSpec(
        num_scalar_prefetch=3,
        grid=(n_blocks,),
        in_specs=[
            pl.BlockSpec((BM, I), lambda i, be, rid, nxe: (i, 0)),
            pl.BlockSpec(memory_space=pl.ANY),
        ],
        out_specs=pl.BlockSpec((BM, D), lambda i, be, rid, nxe: (i, 0)),
        scratch_shapes=[pltpu.VMEM((2, D, I), jnp.float32),
                        pltpu.VMEM((D, I), jnp.bfloat16),
                        pltpu.SemaphoreType.DMA((2,))],
    )
    y_pad = pl.pallas_call(
        _down_body,
        grid_spec=grid_d,
        compiler_params=pltpu.CompilerParams(
            dimension_semantics=("parallel",)),
        out_shape=jax.ShapeDtypeStruct((N_pad, D), jnp.float32),
    )(block_expert, rid, nxe, h_pad, down_proj)

    w_bc = jnp.broadcast_to(top_k_weights.reshape(-1)[:, None], (T * K, 16))
    return _make_combine(T, D, K, N_pad)(y_pad, dest_slots, w_bc)


# speedup vs baseline: 1.0946x; 1.0208x over previous
import functools

import jax
import jax.numpy as jnp
from jax import lax
from jax.experimental import pallas as pl
from jax.experimental.pallas import tpu as pltpu
from jax.experimental.pallas import tpu_sc as plsc

BM = 256
CR = 16
CT = 8
UNROLL = 8

def _sc_dims():
    try:
        info = plsc.get_sparse_core_info()
        return info.num_cores, info.num_subcores
    except Exception:
        return 2, 16

_NC, _NS = _sc_dims()
_NW = _NC * _NS


def _routing_metadata(top_k_index, E, n_blocks):
    T, K = top_k_index.shape
    flat_e = top_k_index.reshape(-1).astype(jnp.int32)
    onehot = (flat_e[:, None] == jnp.arange(E, dtype=jnp.int32)[None, :])
    onehot = onehot.astype(jnp.int32)
    csum = jnp.cumsum(onehot, axis=0)
    counts = csum[-1]
    rank = jnp.sum(onehot * csum, axis=1) - 1
    padded = ((counts + BM - 1) // BM) * BM
    p_off = jnp.concatenate(
        [jnp.zeros((1,), jnp.int32), jnp.cumsum(padded)[:-1].astype(jnp.int32)])
    dest = jnp.sum(onehot * p_off[None, :], axis=1).astype(jnp.int32) + rank
    blk_end = (jnp.cumsum(padded) // BM).astype(jnp.int32)
    blk_ids = jnp.arange(n_blocks, dtype=jnp.int32)
    block_expert = jnp.sum(
        (blk_end[None, :] <= blk_ids[:, None]).astype(jnp.int32), axis=1)
    block_expert = jnp.minimum(block_expert, E - 1)
    return dest, block_expert


def _make_dispatch(T, D, K, N_pad):
    mesh = plsc.VectorSubcoreMesh(core_axis_name="c", subcore_axis_name="s")
    tpw = T // _NW

    nch = tpw // CR
    assert nch % 2 == 0

    @functools.partial(
        pl.kernel, mesh=mesh,
        out_type=jax.ShapeDtypeStruct((N_pad, D), jnp.float32),
        scratch_types=[
            pltpu.VMEM((CR, D), jnp.float32),
            pltpu.VMEM((CR, D), jnp.float32),
            pltpu.VMEM((1, CR), jnp.int32),
            pltpu.VMEM((1, CR), jnp.int32),
            pltpu.SemaphoreType.DMA, pltpu.SemaphoreType.DMA,
            pltpu.SemaphoreType.DMA, pltpu.SemaphoreType.DMA,
        ],
    )
    def dispatch(hs_hbm, dslot_hbm, xpad_hbm, xa, xb, da, db, ra, rb, sa, sb):
        wid = lax.axis_index("s") * _NC + lax.axis_index("c")
        xr = [xa, xb]
        dbufs = [da, db]
        rsems = [ra, rb]
        ssems = [sa, sb]

        def rd(s, j, par):
            base = wid * tpw + j * CR
            return (hs_hbm.at[pl.ds(base, CR)], xr[par], rsems[par])

        for s in range(K):
            def issue_read(j, par, s=s):
                pltpu.async_copy(*rd(s, j, par))

            def wait_read(j, par, s=s):
                pltpu.make_async_copy(*rd(s, j, par)).wait()

            def scat(j, par, s=s):
                base = wid * tpw + j * CR
                pltpu.sync_copy(dslot_hbm.at[s, pl.ds(base, CR)],
                                dbufs[par].at[0])
                pltpu.async_copy(xr[par], xpad_hbm.at[dbufs[par].at[0]],
                                 ssems[par])

            def wait_scat(j, par, s=s):
                base = wid * tpw + j * CR
                pltpu.make_async_copy(
                    xr[par], xpad_hbm.at[dbufs[par].at[0]], ssems[par]).wait()

            issue_read(0, 0)

            @pl.loop(0, nch, step=2)
            def _(j, issue_read=issue_read, wait_read=wait_read,
                  scat=scat, wait_scat=wait_scat):
                wait_read(j, 0)
                issue_read(j + 1, 1)
                scat(j, 0)

                @pl.when(j + 2 < nch)
                def _():
                    wait_scat(j, 0)
                    issue_read(j + 2, 0)

                wait_read(j + 1, 1)
                scat(j + 1, 1)
                wait_scat(j + 1, 1)

            wait_scat(nch - 2, 0)

    return dispatch


def _make_combine(T, D, K, N_pad):
    mesh = plsc.VectorSubcoreMesh(core_axis_name="c", subcore_axis_name="s")
    tpw = T // _NW
    nch = tpw // CT
    assert nch % 2 == 0

    scratch = [pltpu.VMEM((CT, D), jnp.float32) for _ in range(2 * K)]
    scratch += [pltpu.VMEM((tpw,), jnp.int32) for _ in range(K)]
    scratch += [pltpu.VMEM((tpw * K, 16), jnp.float32)]
    scratch += [pltpu.SemaphoreType.DMA, pltpu.SemaphoreType.DMA]

    @functools.partial(
        pl.kernel, mesh=mesh,
        out_type=jax.ShapeDtypeStruct((T, D), jnp.float32),
        scratch_types=scratch,
    )
    def combine(y_hbm, dslot_hbm, w_hbm, out_hbm, *rest):
        rbufs = [rest[:K], rest[K:2 * K]]
        ibufs = rest[2 * K:3 * K]
        wbuf = rest[3 * K]
        sems = [rest[3 * K + 1], rest[3 * K + 2]]
        wid = lax.axis_index("s") * _NC + lax.axis_index("c")
        base0 = wid * tpw

        for s in range(K):
            pltpu.sync_copy(dslot_hbm.at[s, pl.ds(base0, tpw)], ibufs[s])
        pltpu.sync_copy(w_hbm.at[pl.ds(base0 * K, tpw * K)], wbuf)

        def issue(j, par):
            for s in range(K):
                pltpu.async_copy(y_hbm.at[ibufs[s].at[pl.ds(j * CT, CT)]],
                                 rbufs[par][s], sems[par])

        def wait(j, par):
            for s in range(K):
                pltpu.make_async_copy(
                    y_hbm.at[ibufs[s].at[pl.ds(j * CT, CT)]],
                    rbufs[par][s], sems[par]).wait()

        def compute_store(j, par):
            bufs = rbufs[par]

            @pl.loop(0, CT)
            def _(t):
                woff = (j * CT + t) * K

                @pl.loop(0, D, step=16 * UNROLL)
                def _(v):
                    for u in range(UNROLL):
                        sl = pl.ds(v + u * 16, 16)
                        acc = wbuf[woff] * bufs[0][t, sl]
                        for s in range(1, K):
                            acc = acc + wbuf[woff + s] * bufs[s][t, sl]
                        bufs[0][t, sl] = acc

            pltpu.sync_copy(bufs[0], out_hbm.at[pl.ds(base0 + j * CT, CT)])

        issue(0, 0)

        @pl.loop(0, nch, step=2)
        def _(j):
            issue(j + 1, 1)
            wait(j, 0)
            compute_store(j, 0)

            @pl.when(j + 2 < nch)
            def _():
                issue(j + 2, 0)

            wait(j + 1, 1)
            compute_store(j + 1, 1)

    return combine


_DIMS_NT = (((1,), (1,)), ((), ()))


def _run_metadata(block_expert, n_blocks):
    be = block_expert
    chg = jnp.concatenate(
        [jnp.ones((1,), jnp.int32), (be[1:] != be[:-1]).astype(jnp.int32)])
    rid = jnp.cumsum(chg) - 1
    run_expert = jnp.zeros((n_blocks + 1,), jnp.int32).at[rid].set(be)
    n_runs = rid[-1] + 1
    nxe = jnp.where(rid + 1 < n_runs, run_expert[rid + 1], -1)
    return rid.astype(jnp.int32), nxe.astype(jnp.int32)


def _slab(w_any, e, half, span):
    if half is None:
        return w_any.at[e]
    return w_any.at[e, pl.ds(half * span, span)]


def _stage_weights(refs, w_any, half, span):
    be_ref, rid_ref, nxe_ref, stg_ref, wbf_ref, sems = refs
    i = pl.program_id(0)
    rid = rid_ref[i]
    par = lax.rem(rid, 2)
    prev_rid = rid_ref[jnp.maximum(i - 1, 0)]
    is_start = (i == 0) | (rid != prev_rid)

    @pl.when(i == 0)
    def _():
        pltpu.make_async_copy(_slab(w_any, be_ref[0], half, span),
                              stg_ref.at[0], sems.at[0]).start()

    @pl.when(is_start)
    def _():
        pltpu.make_async_copy(_slab(w_any, be_ref[i], half, span),
                              stg_ref.at[par], sems.at[par]).wait()
        wbf_ref[...] = stg_ref[par].astype(jnp.bfloat16)

        @pl.when(nxe_ref[i] >= 0)
        def _():
            pltpu.make_async_copy(_slab(w_any, nxe_ref[i], half, span),
                                  stg_ref.at[1 - par], sems.at[1 - par]).start()


def _gate_body(be_ref, rid_ref, nxe_ref, x_ref, w_any, sg_ref, xbf_ref,
               stg_ref, wbf_ref, sems):
    _stage_weights((be_ref, rid_ref, nxe_ref, stg_ref, wbf_ref, sems),
                   w_any, 0, wbf_ref.shape[0])
    x = x_ref[...].astype(jnp.bfloat16)
    xbf_ref[...] = x
    g = lax.dot_general(x, wbf_ref[...], _DIMS_NT,
                        preferred_element_type=jnp.float32)
    sg_ref[...] = (g * jax.nn.sigmoid(g)).astype(jnp.bfloat16)


def _up_body(be_ref, rid_ref, nxe_ref, xbf_ref, w_any, sg_ref, h_ref,
             stg_ref, wbf_ref, sems):
    _stage_weights((be_ref, rid_ref, nxe_ref, stg_ref, wbf_ref, sems),
                   w_any, 1, wbf_ref.shape[0])
    up = lax.dot_general(xbf_ref[...], wbf_ref[...], _DIMS_NT,
                         preferred_element_type=jnp.float32)
    h_ref[...] = (up * sg_ref[...].astype(jnp.float32)).astype(jnp.bfloat16)


def _down_body(be_ref, rid_ref, nxe_ref, h_ref, w_any, y_ref,
               stg_ref, wbf_ref, sems):
    _stage_weights((be_ref, rid_ref, nxe_ref, stg_ref, wbf_ref, sems),
                   w_any, None, None)
    y_ref[...] = lax.dot_general(h_ref[...], wbf_ref[...], _DIMS_NT,
                                 preferred_element_type=jnp.float32)


def kernel(hidden_states, top_k_index, top_k_weights, gate_up_proj, down_proj):
    T, D = hidden_states.shape
    K = top_k_index.shape[1]
    E, I2, _ = gate_up_proj.shape
    I = I2 // 2
    N_pad = T * K + E * BM
    n_blocks = N_pad // BM

    dest, block_expert = _routing_metadata(top_k_index, E, n_blocks)
    dest_slots = dest.reshape(T, K).T

    x_pad = _make_dispatch(T, D, K, N_pad)(hidden_states, dest_slots)

    rid, nxe = _run_metadata(block_expert, n_blocks)

    grid_g = pltpu.PrefetchScalarGridSpec(
        num_scalar_prefetch=3,
        grid=(n_blocks,),
        in_specs=[
            pl.BlockSpec((BM, D), lambda i, be, rid, nxe: (i, 0)),
            pl.BlockSpec(memory_space=pl.ANY),
        ],
        out_specs=[
            pl.BlockSpec((BM, I), lambda i, be, rid, nxe: (i, 0)),
            pl.BlockSpec((BM, D), lambda i, be, rid, nxe: (i, 0)),
        ],
        scratch_shapes=[pltpu.VMEM((2, I, D), jnp.float32),
                        pltpu.VMEM((I, D), jnp.bfloat16),
                        pltpu.SemaphoreType.DMA((2,))],
    )
    sg_pad, xbf_pad = pl.pallas_call(
        _gate_body,
        grid_spec=grid_g,
        compiler_params=pltpu.CompilerParams(
            dimension_semantics=("parallel",)),
        out_shape=[jax.ShapeDtypeStruct((N_pad, I), jnp.bfloat16),
                   jax.ShapeDtypeStruct((N_pad, D), jnp.bfloat16)],
    )(block_expert, rid, nxe, x_pad, gate_up_proj)

    grid_u = pltpu.PrefetchScalarGridSpec(
        num_scalar_prefetch=3,
        grid=(n_blocks,),
        in_specs=[
            pl.BlockSpec((BM, D), lambda i, be, rid, nxe: (i, 0)),
            pl.BlockSpec(memory_space=pl.ANY),
            pl.BlockSpec((BM, I), lambda i, be, rid, nxe: (i, 0)),
        ],
        out_specs=pl.BlockSpec((BM, I), lambda i, be, rid, nxe: (i, 0)),
        scratch_shapes=[pltpu.VMEM((2, I, D), jnp.float32),
                        pltpu.VMEM((I, D), jnp.bfloat16),
                        pltpu.SemaphoreType.DMA((2,))],
    )
    h_pad = pl.pallas_call(
        _up_body,
        grid_spec=grid_u,
        compiler_params=pltpu.CompilerParams(
            dimension_semantics=("parallel",)),
        out_shape=jax.ShapeDtypeStruct((N_pad, I), jnp.bfloat16),
    )(block_expert, rid, nxe, xbf_pad, gate_up_proj, sg_pad)

    grid_d = pltpu.PrefetchScalarGridSpec(
        num_scalar_prefetch=3,
        grid=(n_blocks,),
        in_specs=[
            pl.BlockSpec((BM, I), lambda i, be, rid, nxe: (i, 0)),
            pl.BlockSpec(memory_space=pl.ANY),
        ],
        out_specs=pl.BlockSpec((BM, D), lambda i, be, rid, nxe: (i, 0)),
        scratch_shapes=[pltpu.VMEM((2, D, I), jnp.float32),
                        pltpu.VMEM((D, I), jnp.bfloat16),
                        pltpu.SemaphoreType.DMA((2,))],
    )
    y_pad = pl.pallas_call(
        _down_body,
        grid_spec=grid_d,
        compiler_params=pltpu.CompilerParams(
            dimension_semantics=("parallel",)),
        out_shape=jax.ShapeDtypeStruct((N_pad, D), jnp.float32),
    )(block_expert, rid, nxe, h_pad, down_proj)

    w_bc = jnp.broadcast_to(top_k_weights.reshape(-1)[:, None], (T * K, 16))
    return _make_combine(T, D, K, N_pad)(y_pad, dest_slots, w_bc)
